# baseline (speedup 1.0000x reference)
import jax
import jax.numpy as jnp
from jax import lax
from jax.experimental import pallas as pl
from jax.experimental.pallas import tpu as pltpu, tpu_sc as plsc

B = 4096
L = 50
D = 64
LANES = 16
NC, NS = 2, 16
NW = NC * NS
GROUPS = B // (NW * LANES)
IDS_PER_GROUP = LANES * L
ROUNDS = 8
IDS_PER_ROUND = IDS_PER_GROUP // ROUNDS
NGROUP = B // LANES


def _sc_body(ids_hbm, w_hbm, table_hbm, out_hbm, idx_v, w_v, rows_v, out_v, sem):
    cid = lax.axis_index("c")
    sid = lax.axis_index("s")
    wid = sid * NC + cid

    iota = jax.lax.iota(jnp.int32, LANES)
    row_base = iota * L

    def group_body(g, carry):
        gidx = wid * GROUPS + g

        pltpu.sync_copy(ids_hbm.at[gidx], idx_v)
        pltpu.sync_copy(w_hbm.at[gidx], w_v)

        copies = []
        for r in range(ROUNDS):
            copies.append(
                pltpu.async_copy(
                    table_hbm.at[idx_v.at[r]],
                    rows_v.at[pl.ds(r * IDS_PER_ROUND, IDS_PER_ROUND)],
                    sem,
                )
            )
        for c in copies:
            c.wait()

        for d_blk in range(D // LANES):
            zeros = tuple(jnp.zeros((LANES,), jnp.float32) for _ in range(LANES))

            def j_body(j, accs, _d_blk=d_blk):
                w_vec = w_v[j, :]
                idx_row = row_base + j
                out = []
                for d in range(LANES):
                    col = jnp.full((LANES,), _d_blk * LANES + d, jnp.int32)
                    vals = plsc.load_gather(rows_v, [idx_row, col])
                    out.append(accs[d] + w_vec * vals)
                return tuple(out)

            accs = lax.fori_loop(0, L, j_body, zeros)
            for d in range(LANES):
                col = jnp.full((LANES,), d_blk * LANES + d, jnp.int32)
                plsc.store_scatter(out_v, [iota, col], accs[d])

        pltpu.sync_copy(out_v, out_hbm.at[pl.ds(gidx * LANES, LANES)])
        return carry

    lax.fori_loop(0, GROUPS, group_body, 0)


_sc_kernel = pl.kernel(
    _sc_body,
    out_type=jax.ShapeDtypeStruct((B, D), jnp.float32),
    mesh=plsc.VectorSubcoreMesh(core_axis_name="c", subcore_axis_name="s"),
    scratch_types=[
        pltpu.VMEM((ROUNDS, IDS_PER_ROUND), jnp.int32),
        pltpu.VMEM((L, LANES), jnp.float32),
        pltpu.VMEM((IDS_PER_GROUP, D), jnp.float32),
        pltpu.VMEM((LANES, D), jnp.float32),
        pltpu.SemaphoreType.DMA,
    ],
    compiler_params=pltpu.CompilerParams(
        use_tc_tiling_on_sc=False, needs_layout_passes=False
    ),
)


def kernel(sp_ids, sp_weights, embeddings):
    ids_r = sp_ids.reshape(NGROUP, ROUNDS, IDS_PER_ROUND)
    w_r = sp_weights.reshape(NGROUP, LANES, L).transpose(0, 2, 1)
    return _sc_kernel(ids_r, w_r, embeddings)


# speedup vs baseline: 1.0021x; 1.0021x over previous
import jax
import jax.numpy as jnp
from jax import lax
from jax.experimental import pallas as pl
from jax.experimental.pallas import tpu as pltpu, tpu_sc as plsc

B = 4096
L = 50
D = 64
LANES = 16
NC, NS = 2, 16
NW = NC * NS
GROUPS = B // (NW * LANES)
IDS_PER_GROUP = LANES * L


def _sc_body(ids_hbm, w_hbm, table_hbm, out_hbm, idx_v, w_v, rows_v, out_v, sem):
    cid = lax.axis_index("c")
    sid = lax.axis_index("s")
    wid = sid * NC + cid

    iota = jax.lax.iota(jnp.int32, LANES)
    row_base = iota * L

    def group_body(g, carry):
        gidx = wid * GROUPS + g
        b0 = gidx * LANES

        pltpu.sync_copy(ids_hbm.at[pl.ds(b0, LANES)], idx_v)
        pltpu.sync_copy(w_hbm.at[pl.ds(b0, LANES)], w_v)

        copies = []
        for b in range(LANES):
            copies.append(
                pltpu.async_copy(
                    table_hbm.at[idx_v.at[b]],
                    rows_v.at[pl.ds(b * L, L)],
                    sem,
                )
            )
        for c in copies:
            c.wait()

        for d_blk in range(D // LANES):
            zeros = tuple(jnp.zeros((LANES,), jnp.float32) for _ in range(LANES))

            def j_body(j, accs, _d_blk=d_blk):
                jcol = jnp.full((LANES,), 0, jnp.int32) + j
                w_vec = plsc.load_gather(w_v, [iota, jcol])
                idx_row = row_base + j
                out = []
                for d in range(LANES):
                    col = jnp.full((LANES,), _d_blk * LANES + d, jnp.int32)
                    vals = plsc.load_gather(rows_v, [idx_row, col])
                    out.append(accs[d] + w_vec * vals)
                return tuple(out)

            accs = lax.fori_loop(0, L, j_body, zeros)
            for d in range(LANES):
                col = jnp.full((LANES,), d_blk * LANES + d, jnp.int32)
                plsc.store_scatter(out_v, [iota, col], accs[d])

        pltpu.sync_copy(out_v, out_hbm.at[pl.ds(b0, LANES)])
        return carry

    lax.fori_loop(0, GROUPS, group_body, 0)


_sc_kernel = pl.kernel(
    _sc_body,
    out_type=jax.ShapeDtypeStruct((B, D), jnp.float32),
    mesh=plsc.VectorSubcoreMesh(core_axis_name="c", subcore_axis_name="s"),
    scratch_types=[
        pltpu.VMEM((LANES, L), jnp.int32),
        pltpu.VMEM((LANES, L), jnp.float32),
        pltpu.VMEM((IDS_PER_GROUP, D), jnp.float32),
        pltpu.VMEM((LANES, D), jnp.float32),
        pltpu.SemaphoreType.DMA,
    ],
    compiler_params=pltpu.CompilerParams(
        use_tc_tiling_on_sc=False, needs_layout_passes=False
    ),
)


def kernel(sp_ids, sp_weights, embeddings):
    return _sc_kernel(sp_ids, sp_weights, embeddings)


# speedup vs baseline: 1.3424x; 1.3396x over previous
import jax
import jax.numpy as jnp
from jax import lax
from jax.experimental import pallas as pl
from jax.experimental.pallas import tpu as pltpu, tpu_sc as plsc

B = 4096
L = 50
D = 64
DPAD = 128
LANES = 16
NC, NS = 2, 16
NW = NC * NS
GROUPS = B // (NW * LANES)
IDS_PER_GROUP = LANES * L


LH = L // 2
WPAD = 64


def _sc_body(ids_hbm, w_hbm, table_hbm, out_hbm, idx0, idx1, w0, w1,
             rows_a, rows_b, out0, out1, sem_a, sem_b, sem_s, sem_o):
    cid = lax.axis_index("c")
    sid = lax.axis_index("s")
    wid = sid * NC + cid

    idx_bufs = (idx0, idx1)
    w_bufs = (w0, w1)
    out_bufs = (out0, out1)

    def stage(g, sync):
        b0 = (wid * GROUPS + g) * LANES
        p = g % 2
        if sync:
            pltpu.sync_copy(ids_hbm.at[pl.ds(b0, LANES)], idx_bufs[p])
            pltpu.sync_copy(w_hbm.at[pl.ds(b0, LANES)], w_bufs[p])
            return ()
        return (
            pltpu.async_copy(ids_hbm.at[pl.ds(b0, LANES)], idx_bufs[p], sem_s),
            pltpu.async_copy(w_hbm.at[pl.ds(b0, LANES)], w_bufs[p], sem_s),
        )

    def fire(g, half, rows_buf, sem):
        idx_v = idx_bufs[g % 2]
        return [
            pltpu.async_copy(
                table_hbm.at[idx_v.at[b, pl.ds(half * LH, LH)]],
                rows_buf.at[pl.ds(b * LH, LH), :],
                sem,
            )
            for b in range(LANES)
        ]

    def accumulate(g, half, rows_buf):
        w_v = w_bufs[g % 2]
        out_v = out_bufs[g % 2]

        def b_body(b, carry):
            chunks = {
                c: w_v[b, pl.ds(16 * c, 16)]
                for c in range((half * LH) // 16, (half * LH + LH - 1) // 16 + 1)
            }
            accs = [jnp.zeros((LANES,), jnp.float32) for _ in range(D // LANES)]
            for j_local in range(LH):
                j_abs = half * LH + j_local
                lane = jnp.full((LANES,), j_abs % 16, jnp.int32)
                wb = lax.gather(
                    chunks[j_abs // 16], lane[:, None],
                    dimension_numbers=lax.GatherDimensionNumbers(
                        offset_dims=(), collapsed_slice_dims=(0,),
                        start_index_map=(0,)),
                    slice_sizes=(1,),
                    mode=lax.GatherScatterMode.PROMISE_IN_BOUNDS)
                row = b * LH + j_local
                for db in range(D // LANES):
                    vals = rows_buf[row, pl.ds(db * 16, 16)]
                    accs[db] = accs[db] + wb * vals
            for db in range(D // LANES):
                if half == 0:
                    out_v[b, pl.ds(db * 16, 16)] = accs[db]
                else:
                    plsc.addupdate(out_v.at[b, pl.ds(db * 16, 16)], accs[db])
            return carry

        lax.fori_loop(0, LANES, b_body, 0)

    stage(0, sync=True)
    copies_a = fire(0, 0, rows_a, sem_a)
    copies_b = fire(0, 1, rows_b, sem_b)
    out_copies = [None, None]
    for g in range(GROUPS):
        staging = stage(g + 1, sync=False) if g + 1 < GROUPS else ()
        for c in copies_a:
            c.wait()
        if out_copies[g % 2] is not None:
            out_copies[g % 2].wait()
            out_copies[g % 2] = None
        accumulate(g, 0, rows_a)
        for c in copies_b:
            c.wait()
        if g + 1 < GROUPS:
            for c in staging:
                c.wait()
            copies_a = fire(g + 1, 0, rows_a, sem_a)
        accumulate(g, 1, rows_b)
        if g + 1 < GROUPS:
            copies_b = fire(g + 1, 1, rows_b, sem_b)
        b0 = (wid * GROUPS + g) * LANES
        out_copies[g % 2] = pltpu.async_copy(
            out_bufs[g % 2], out_hbm.at[pl.ds(b0, LANES)], sem_o)
    for c in out_copies:
        if c is not None:
            c.wait()


_sc_kernel = pl.kernel(
    _sc_body,
    out_type=jax.ShapeDtypeStruct((B, D), jnp.float32),
    mesh=plsc.VectorSubcoreMesh(core_axis_name="c", subcore_axis_name="s"),
    scratch_types=[
        pltpu.VMEM((LANES, L), jnp.int32),
        pltpu.VMEM((LANES, L), jnp.int32),
        pltpu.VMEM((LANES, WPAD), jnp.float32),
        pltpu.VMEM((LANES, WPAD), jnp.float32),
        pltpu.VMEM((IDS_PER_GROUP // 2, DPAD), jnp.float32),
        pltpu.VMEM((IDS_PER_GROUP // 2, DPAD), jnp.float32),
        pltpu.VMEM((LANES, D), jnp.float32),
        pltpu.VMEM((LANES, D), jnp.float32),
        pltpu.SemaphoreType.DMA,
        pltpu.SemaphoreType.DMA,
        pltpu.SemaphoreType.DMA,
        pltpu.SemaphoreType.DMA,
    ],
    compiler_params=pltpu.CompilerParams(
        use_tc_tiling_on_sc=True, needs_layout_passes=False
    ),
)


def kernel(sp_ids, sp_weights, embeddings):
    emb_pad = jnp.pad(embeddings, ((0, 0), (0, DPAD - D)))
    w_pad = jnp.pad(sp_weights, ((0, 0), (0, WPAD - L)))
    return _sc_kernel(sp_ids, w_pad, emb_pad)
